# Initial kernel scaffold; baseline (speedup 1.0000x reference)
import jax
import jax.numpy as jnp
from jax.experimental import pallas as pl


def kernel(sample_weight, target_idx):
    raise NotImplementedError("write your pallas kernel here")


# speedup vs baseline: 1.6320x; 1.6320x over previous
import jax
import jax.numpy as jnp
from jax.experimental import pallas as pl
from jax.experimental.pallas import tpu as pltpu

_TAU = 0.01
_N_EDGES = 4
_BR = 8


def _sample_body(tgt_ref, w_ref, g0_ref, g1_ref, g2_ref, out_ref):
    w = w_ref[...]
    tgt = tgt_ref[0]
    cols = jax.lax.broadcasted_iota(jnp.int32, w.shape, 1)
    neg = jnp.float32(-3e38)
    big = jnp.int32(2**30)

    def argmax_rows(s):
        m = jnp.max(s, axis=1, keepdims=True)
        return jnp.min(jnp.where(s >= m, cols, big), axis=1)

    mask = cols == tgt
    m0 = argmax_rows(jnp.where(mask, neg, w + g0_ref[...]))
    mask = mask | (cols == m0[:, None])
    m1 = argmax_rows(jnp.where(mask, neg, w + g1_ref[...]))
    mask = mask | (cols == m1[:, None])
    m2 = argmax_rows(jnp.where(mask, neg, w + g2_ref[...]))

    out_ref[:, 0] = jnp.full((_BR,), tgt, jnp.float32)
    out_ref[:, 1] = m0.astype(jnp.float32)
    out_ref[:, 2] = m1.astype(jnp.float32)
    out_ref[:, 3] = m2.astype(jnp.float32)


def kernel(sample_weight, target_idx):
    num_nodes, num_targets = sample_weight.shape
    skey = jax.random.key(42)
    g0, g1, g2 = (
        jax.random.gumbel(jax.random.fold_in(skey, i),
                          (num_nodes, num_targets), jnp.float32)
        for i in range(_N_EDGES - 1)
    )
    tgt = jnp.asarray(target_idx, jnp.int32).reshape(1)

    grid = (num_nodes // _BR,)
    row_spec = pl.BlockSpec((_BR, num_targets), lambda i, *_: (i, 0))
    out = pl.pallas_call(
        _sample_body,
        grid_spec=pltpu.PrefetchScalarGridSpec(
            num_scalar_prefetch=1,
            grid=grid,
            in_specs=[row_spec, row_spec, row_spec, row_spec],
            out_specs=pl.BlockSpec((_BR, _N_EDGES), lambda i, *_: (i, 0)),
        ),
        out_shape=jax.ShapeDtypeStruct((num_nodes, _N_EDGES), jnp.float32),
    )(tgt, sample_weight, g0, g1, g2)
    return out
